# baseline (speedup 1.0000x reference)
import functools

import jax
import jax.numpy as jnp
from jax import lax
from jax.experimental import pallas as pl
from jax.experimental.pallas import tpu as pltpu
from jax.experimental.pallas import tpu_sc as plsc

BATCH = 16384
EMBED = 32
NUM_CORES = 2
NUM_SUBCORES = 16
NUM_WORKERS = NUM_CORES * NUM_SUBCORES
BPW = BATCH // NUM_WORKERS
GCHUNK = 128

_mesh = plsc.VectorSubcoreMesh(core_axis_name="c", subcore_axis_name="s")


@functools.partial(
    pl.kernel,
    out_type=[
        jax.ShapeDtypeStruct((BATCH, EMBED), jnp.float32),
        jax.ShapeDtypeStruct((BATCH, EMBED), jnp.float32),
    ],
    mesh=_mesh,
    scratch_types=[
        pltpu.VMEM((BPW,), jnp.int32),
        pltpu.VMEM((BPW,), jnp.int32),
        pltpu.VMEM((BPW, EMBED), jnp.float32),
        pltpu.VMEM((BPW, EMBED), jnp.float32),
        pltpu.SemaphoreType.DMA,
        pltpu.SemaphoreType.DMA,
    ],
    compiler_params=pltpu.CompilerParams(use_tc_tiling_on_sc=False),
)
def _sc_gather(uids_hbm, iids_hbm, utab_hbm, itab_hbm, uout_hbm, iout_hbm,
               uidx, iidx, urows, irows, gsem, osem):
    wid = lax.axis_index("s") * NUM_CORES + lax.axis_index("c")
    base = wid * BPW
    pltpu.sync_copy(uids_hbm.at[pl.ds(base, BPW)], uidx)
    pltpu.sync_copy(iids_hbm.at[pl.ds(base, BPW)], iidx)
    copies = []
    for k in range(BPW // GCHUNK):
        sl = pl.ds(k * GCHUNK, GCHUNK)
        copies.append(
            pltpu.async_copy(utab_hbm.at[uidx.at[sl]], urows.at[sl], gsem))
        copies.append(
            pltpu.async_copy(itab_hbm.at[iidx.at[sl]], irows.at[sl], gsem))
    for c in copies:
        c.wait()
    ou = pltpu.async_copy(urows, uout_hbm.at[pl.ds(base, BPW)], osem)
    oi = pltpu.async_copy(irows, iout_hbm.at[pl.ds(base, BPW)], osem)
    ou.wait()
    oi.wait()


_BB = 2048


def _combine_body(u_ref, i_ref, g_ref, a_ref, gw_ref, gb_ref, aw_ref, ab_ref,
                  o_ref):
    pa = jnp.dot(a_ref[...], aw_ref[...],
                 preferred_element_type=jnp.float32) + ab_ref[...]
    pg = jnp.dot(g_ref[...], gw_ref[...],
                 preferred_element_type=jnp.float32) + gb_ref[...]
    s = jnp.sum((u_ref[...] + pa) * (i_ref[...] + pg), axis=1)
    o_ref[...] = jax.nn.sigmoid(s) * 4.0 + 1.0


_tc_combine = pl.pallas_call(
    _combine_body,
    grid=(BATCH // _BB,),
    in_specs=[
        pl.BlockSpec((_BB, EMBED), lambda i: (i, 0)),
        pl.BlockSpec((_BB, EMBED), lambda i: (i, 0)),
        pl.BlockSpec((_BB, EMBED), lambda i: (i, 0)),
        pl.BlockSpec((_BB, 8), lambda i: (i, 0)),
        pl.BlockSpec((EMBED, EMBED), lambda i: (0, 0)),
        pl.BlockSpec((1, EMBED), lambda i: (0, 0)),
        pl.BlockSpec((8, EMBED), lambda i: (0, 0)),
        pl.BlockSpec((1, EMBED), lambda i: (0, 0)),
    ],
    out_specs=pl.BlockSpec((_BB,), lambda i: (i,)),
    out_shape=jax.ShapeDtypeStruct((BATCH,), jnp.float32),
)


def kernel(user_ids, item_ids, genres_features, age_features, user_table,
           item_table, genres_W, genres_b, age_W, age_b, age_weight,
           genre_weight):
    urows, irows = _sc_gather(user_ids, item_ids, user_table, item_table)
    gw = genre_weight[0] * genres_W
    gb = (genre_weight[0] * genres_b)[None, :]
    aw = age_weight[0] * age_W
    ab = (age_weight[0] * age_b)[None, :]
    return _tc_combine(urows, irows, genres_features, age_features, gw, gb,
                       aw, ab)
